# baseline (speedup 1.0000x reference)
import functools
import math

import numpy as np

import jax
import jax.numpy as jnp
from jax import lax
from jax.experimental import pallas as pl
from jax.experimental.pallas import tpu as pltpu
from jax.experimental.pallas import tpu_sc as plsc

_N = 10000
_NP = 10240
_E = 320000
_EP = 327680
_EPD = 344064
_H = 128
_L = 64
_EPS = 1e-05
_TILES = 32
_CHUNK = 128
_NCHUNK = _EP // _TILES // _CHUNK
_ROWS_PER_SUB = _NP // 16
_DUMMY = 10008

_sc_mesh = plsc.VectorSubcoreMesh(core_axis_name="c", subcore_axis_name="s")

_PERM = np.concatenate(
    [32 * k + np.concatenate([np.arange(0, 32, 2), np.arange(1, 32, 2)])
     for k in range(4)])


_GRP = 8
_CH_A = 160
_CH_B = 8


def _sc_agg_body(table, src2d, dst2d, zrows, out, agg_sh, sblk, dblk,
                 rows0, rows1, g0, g1):
    c = lax.axis_index("c")
    s = lax.axis_index("s")
    rows = (rows0, rows1)
    gsems = (g0, g1)
    nch = jnp.where(c == 0, _CH_A, _CH_B)
    base = jnp.where(c == 0, s * _CH_A, 16 * _CH_A + s * _CH_B)
    ngrp = nch // _GRP
    pltpu.sync_copy(zrows, agg_sh.at[pl.ds(s * _ROWS_PER_SUB, _ROWS_PER_SUB)])
    plsc.subcore_barrier()

    def grp(g, carry):
        pltpu.sync_copy(src2d.at[pl.ds(base + g * _GRP, _GRP)], sblk)
        pltpu.sync_copy(dst2d.at[pl.ds(base + g * _GRP, _GRP)], dblk)
        pltpu.async_copy(table.at[sblk.at[0]], rows0, g0)
        pltpu.async_copy(table.at[sblk.at[1]], rows1, g1)
        for j in range(_GRP):
            b = j % 2
            pltpu.make_async_copy(table.at[sblk.at[j]], rows[b],
                                  gsems[b]).wait()
            pltpu.sync_copy(rows[b], agg_sh.at[dblk.at[j]], add=True)
            if j + 2 < _GRP:
                pltpu.async_copy(table.at[sblk.at[j + 2]], rows[b], gsems[b])
        return carry

    lax.fori_loop(0, ngrp, grp, 0)
    plsc.subcore_barrier()
    pltpu.sync_copy(agg_sh.at[pl.ds(s * _ROWS_PER_SUB, _ROWS_PER_SUB)],
                    out.at[c, pl.ds(s * _ROWS_PER_SUB, _ROWS_PER_SUB)])


_sc_agg = pl.kernel(
    _sc_agg_body,
    out_type=jax.ShapeDtypeStruct((2, _NP, _H), jnp.float32),
    mesh=_sc_mesh,
    scratch_types=[
        pltpu.VMEM_SHARED((_NP, _H), jnp.float32),
        pltpu.VMEM((_GRP, _CHUNK), jnp.int32),
        pltpu.VMEM((_GRP, _CHUNK), jnp.int32),
        pltpu.VMEM((_CHUNK, _H), jnp.float32),
        pltpu.VMEM((_CHUNK, _H), jnp.float32),
        pltpu.SemaphoreType.DMA,
        pltpu.SemaphoreType.DMA,
    ],
)


def _sc_cnt_body(dst2d, ones, zrows, out, cnt_sh, dst_v, ones_v):
    c = lax.axis_index("c")
    s = lax.axis_index("s")
    wid = s * 2 + c
    pltpu.sync_copy(zrows, cnt_sh.at[pl.ds(s * _ROWS_PER_SUB, _ROWS_PER_SUB)])
    pltpu.sync_copy(dst2d.at[pl.ds(wid * _NCHUNK, _NCHUNK)], dst_v)
    pltpu.sync_copy(ones, ones_v)
    plsc.subcore_barrier()

    def chunk(i, carry):
        pltpu.sync_copy(ones_v, cnt_sh.at[dst_v.at[i]], add=True)
        return carry

    lax.fori_loop(0, _NCHUNK, chunk, 0)
    plsc.subcore_barrier()
    pltpu.sync_copy(cnt_sh.at[pl.ds(s * _ROWS_PER_SUB, _ROWS_PER_SUB)],
                    out.at[c, pl.ds(s * _ROWS_PER_SUB, _ROWS_PER_SUB)])


_sc_cnt = pl.kernel(
    _sc_cnt_body,
    out_type=jax.ShapeDtypeStruct((2, _NP, _H), jnp.float32),
    mesh=_sc_mesh,
    scratch_types=[
        pltpu.VMEM_SHARED((_NP, _H), jnp.float32),
        pltpu.VMEM((_NCHUNK, _CHUNK), jnp.int32),
        pltpu.VMEM((_CHUNK, _H), jnp.float32),
    ],
)


_BLK = 2560
_RSQRT2 = 1.0 / math.sqrt(2.0)
_BN_SCALE = 1.0 / math.sqrt(1.0 + _EPS)


def _tc_layer_body(p0, p1, c0, c1, h, wl, bl, wr, g, be, o):
    i = pl.program_id(0)
    cnt = c0[:, 0:1] + c1[:, 0:1]
    mean = (p0[...] + p1[...]) / jnp.maximum(cnt, 1.0)
    t = lax.dot_general(mean, wl[...], (((1,), (1,)), ((), ())),
                        preferred_element_type=jnp.float32)
    t = t + bl[...]
    t = t + lax.dot_general(h[...], wr[...], (((1,), (1,)), ((), ())),
                            preferred_element_type=jnp.float32)
    t = (t * _BN_SCALE) * g[...] + be[...]
    a = 0.5 * t * (1.0 + lax.erf(t * _RSQRT2))
    row = i * _BLK + lax.broadcasted_iota(jnp.int32, a.shape, 0)
    o[...] = jnp.where(row < _N, a, 0.0)


_tc_layer = pl.pallas_call(
    _tc_layer_body,
    grid=(4,),
    in_specs=[
        pl.BlockSpec((_BLK, _H), lambda i: (i, 0)),
        pl.BlockSpec((_BLK, _H), lambda i: (i, 0)),
        pl.BlockSpec((_BLK, _H), lambda i: (i, 0)),
        pl.BlockSpec((_BLK, _H), lambda i: (i, 0)),
        pl.BlockSpec((_BLK, _H), lambda i: (i, 0)),
        pl.BlockSpec((_H, _H), lambda i: (0, 0)),
        pl.BlockSpec((1, _H), lambda i: (0, 0)),
        pl.BlockSpec((_H, _H), lambda i: (0, 0)),
        pl.BlockSpec((1, _H), lambda i: (0, 0)),
        pl.BlockSpec((1, _H), lambda i: (0, 0)),
    ],
    out_specs=pl.BlockSpec((_BLK, _H), lambda i: (i, 0)),
    out_shape=jax.ShapeDtypeStruct((_NP, _H), jnp.float32),
)


def _tc_head_body(h, wmu, bmu, wls, bls, wdec, bdec, xo, muo, lso):
    mu = lax.dot_general(h[...], wmu[...], (((1,), (1,)), ((), ())),
                         preferred_element_type=jnp.float32) + bmu[...]
    ls = lax.dot_general(h[...], wls[...], (((1,), (1,)), ((), ())),
                         preferred_element_type=jnp.float32) + bls[...]
    ls = jnp.minimum(ls, 10.0)
    xr = lax.dot_general(mu, wdec[...], (((1,), (1,)), ((), ())),
                         preferred_element_type=jnp.float32) + bdec[...]
    xo[...] = xr
    muo[...] = mu
    lso[...] = ls


_tc_head = pl.pallas_call(
    _tc_head_body,
    grid=(4,),
    in_specs=[
        pl.BlockSpec((_BLK, _H), lambda i: (i, 0)),
        pl.BlockSpec((_L, _H), lambda i: (0, 0)),
        pl.BlockSpec((1, _L), lambda i: (0, 0)),
        pl.BlockSpec((_L, _H), lambda i: (0, 0)),
        pl.BlockSpec((1, _L), lambda i: (0, 0)),
        pl.BlockSpec((_H, _L), lambda i: (0, 0)),
        pl.BlockSpec((1, _H), lambda i: (0, 0)),
    ],
    out_specs=[
        pl.BlockSpec((_BLK, _H), lambda i: (i, 0)),
        pl.BlockSpec((_BLK, _L), lambda i: (i, 0)),
        pl.BlockSpec((_BLK, _L), lambda i: (i, 0)),
    ],
    out_shape=[
        jax.ShapeDtypeStruct((_NP, _H), jnp.float32),
        jax.ShapeDtypeStruct((_NP, _L), jnp.float32),
        jax.ShapeDtypeStruct((_NP, _L), jnp.float32),
    ],
)


def kernel(x, edge_index, Wl0, bl0, Wr0, g0, be0, Wl1, bl1, Wr1, g1, be1,
           Wl2, bl2, Wr2, g2, be2, Wmu, bmu, Wls, bls, Wdec, bdec):
    src = edge_index[0]
    dst = edge_index[1]
    pad = jnp.full((_EPD - _E,), _DUMMY, jnp.int32)
    src2d = jnp.concatenate([src, pad]).reshape(_EPD // _CHUNK, _CHUNK)
    dst2d = jnp.concatenate([dst, pad]).reshape(_EPD // _CHUNK, _CHUNK)

    zrows = jnp.zeros((_ROWS_PER_SUB, _H), jnp.float32)
    ones = jnp.ones((_CHUNK, _H), jnp.float32)

    cnt = _sc_cnt(dst2d, ones, zrows)
    c0 = cnt[0]
    c1 = cnt[1]

    h = jnp.pad(x, ((0, _NP - _N), (0, 0)))
    layers = [(Wl0, bl0, Wr0, g0, be0), (Wl1, bl1, Wr1, g1, be1),
              (Wl2, bl2, Wr2, g2, be2)]
    for (Wl, bl, Wr, g, be) in layers:
        agg = _sc_agg(h, src2d, dst2d, zrows)
        h = _tc_layer(agg[0], agg[1], c0, c1, h, Wl, bl.reshape(1, _H), Wr,
                      g.reshape(1, _H), be.reshape(1, _H))

    xr, mu, ls = _tc_head(h, Wmu, bmu.reshape(1, _L), Wls, bls.reshape(1, _L),
                          Wdec, bdec.reshape(1, _H))
    xr = xr[:_N]
    mu = mu[:_N]
    ls = ls[:_N]
    return (xr, mu, mu, ls)


# speedup vs baseline: 2.5498x; 2.5498x over previous
import functools
import math

import numpy as np

import jax
import jax.numpy as jnp
from jax import lax
from jax.experimental import pallas as pl
from jax.experimental.pallas import tpu as pltpu
from jax.experimental.pallas import tpu_sc as plsc

_N = 10000
_NP = 10240
_E = 320000
_EP = 327680
_H = 128
_L = 64
_EPS = 1e-05
_TILES = 32
_CHUNK = 128
_NCHUNK = _EP // _TILES // _CHUNK
_ROWS_PER_SUB = _NP // 16
_DUMMY = 10008

_sc_mesh = plsc.VectorSubcoreMesh(core_axis_name="c", subcore_axis_name="s")

_PERM = np.concatenate(
    [32 * k + np.concatenate([np.arange(0, 32, 2), np.arange(1, 32, 2)])
     for k in range(4)])


_GRP = 8
_CH_A = 152
_CH_B = 8


def _sc_agg_body(table, src2d, dst2d, zrows, out, agg_sh, sblk, dblk,
                 rows0, rows1, g0, g1):
    c = lax.axis_index("c")
    s = lax.axis_index("s")
    rows = (rows0, rows1)
    gsems = (g0, g1)
    nch = jnp.where(c == 0, _CH_A, _CH_B)
    base = jnp.where(c == 0, s * _CH_A, 16 * _CH_A + s * _CH_B)
    ngrp = nch // _GRP
    pltpu.sync_copy(zrows, agg_sh.at[pl.ds(s * _ROWS_PER_SUB, _ROWS_PER_SUB)])
    plsc.subcore_barrier()

    def grp(g, carry):
        pltpu.sync_copy(src2d.at[pl.ds(base + g * _GRP, _GRP)], sblk)
        pltpu.sync_copy(dst2d.at[pl.ds(base + g * _GRP, _GRP)], dblk)
        pltpu.async_copy(table.at[sblk.at[0]], rows0, g0)
        pltpu.async_copy(table.at[sblk.at[1]], rows1, g1)
        for j in range(_GRP):
            b = j % 2
            pltpu.make_async_copy(table.at[sblk.at[j]], rows[b],
                                  gsems[b]).wait()
            pltpu.sync_copy(rows[b], agg_sh.at[dblk.at[j]], add=True)
            if j + 2 < _GRP:
                pltpu.async_copy(table.at[sblk.at[j + 2]], rows[b], gsems[b])
        return carry

    lax.fori_loop(0, ngrp, grp, 0)
    plsc.subcore_barrier()
    pltpu.sync_copy(agg_sh.at[pl.ds(s * _ROWS_PER_SUB, _ROWS_PER_SUB)],
                    out.at[c, pl.ds(s * _ROWS_PER_SUB, _ROWS_PER_SUB)])


_sc_agg = pl.kernel(
    _sc_agg_body,
    out_type=jax.ShapeDtypeStruct((2, _NP, _H), jnp.float32),
    mesh=_sc_mesh,
    scratch_types=[
        pltpu.VMEM_SHARED((_NP, _H), jnp.float32),
        pltpu.VMEM((_GRP, _CHUNK), jnp.int32),
        pltpu.VMEM((_GRP, _CHUNK), jnp.int32),
        pltpu.VMEM((_CHUNK, _H), jnp.float32),
        pltpu.VMEM((_CHUNK, _H), jnp.float32),
        pltpu.SemaphoreType.DMA,
        pltpu.SemaphoreType.DMA,
    ],
)


def _sc_cnt_body(dst2d, ones, zrows, out, cnt_sh, dst_v, ones_v):
    c = lax.axis_index("c")
    s = lax.axis_index("s")
    wid = s * 2 + c
    pltpu.sync_copy(zrows, cnt_sh.at[pl.ds(s * _ROWS_PER_SUB, _ROWS_PER_SUB)])
    pltpu.sync_copy(dst2d.at[pl.ds(wid * _NCHUNK, _NCHUNK)], dst_v)
    pltpu.sync_copy(ones, ones_v)
    plsc.subcore_barrier()

    def chunk(i, carry):
        pltpu.sync_copy(ones_v, cnt_sh.at[dst_v.at[i]], add=True)
        return carry

    lax.fori_loop(0, _NCHUNK, chunk, 0)
    plsc.subcore_barrier()
    pltpu.sync_copy(cnt_sh.at[pl.ds(s * _ROWS_PER_SUB, _ROWS_PER_SUB)],
                    out.at[c, pl.ds(s * _ROWS_PER_SUB, _ROWS_PER_SUB)])


_sc_cnt = pl.kernel(
    _sc_cnt_body,
    out_type=jax.ShapeDtypeStruct((2, _NP, _H), jnp.float32),
    mesh=_sc_mesh,
    scratch_types=[
        pltpu.VMEM_SHARED((_NP, _H), jnp.float32),
        pltpu.VMEM((_NCHUNK, _CHUNK), jnp.int32),
        pltpu.VMEM((_CHUNK, _H), jnp.float32),
    ],
)


_BLK = 2560
_RSQRT2 = 1.0 / math.sqrt(2.0)
_BN_SCALE = 1.0 / math.sqrt(1.0 + _EPS)


def _tc_layer_body(p0, p1, c0, c1, h, wl, bl, wr, g, be, o):
    i = pl.program_id(0)
    cnt = c0[:, 0:1] + c1[:, 0:1]
    mean = (p0[...] + p1[...]) / jnp.maximum(cnt, 1.0)
    t = lax.dot_general(mean, wl[...], (((1,), (1,)), ((), ())),
                        preferred_element_type=jnp.float32)
    t = t + bl[...]
    t = t + lax.dot_general(h[...], wr[...], (((1,), (1,)), ((), ())),
                            preferred_element_type=jnp.float32)
    t = (t * _BN_SCALE) * g[...] + be[...]
    a = 0.5 * t * (1.0 + lax.erf(t * _RSQRT2))
    row = i * _BLK + lax.broadcasted_iota(jnp.int32, a.shape, 0)
    o[...] = jnp.where(row < _N, a, 0.0)


_tc_layer = pl.pallas_call(
    _tc_layer_body,
    grid=(4,),
    in_specs=[
        pl.BlockSpec((_BLK, _H), lambda i: (i, 0)),
        pl.BlockSpec((_BLK, _H), lambda i: (i, 0)),
        pl.BlockSpec((_BLK, _H), lambda i: (i, 0)),
        pl.BlockSpec((_BLK, _H), lambda i: (i, 0)),
        pl.BlockSpec((_BLK, _H), lambda i: (i, 0)),
        pl.BlockSpec((_H, _H), lambda i: (0, 0)),
        pl.BlockSpec((1, _H), lambda i: (0, 0)),
        pl.BlockSpec((_H, _H), lambda i: (0, 0)),
        pl.BlockSpec((1, _H), lambda i: (0, 0)),
        pl.BlockSpec((1, _H), lambda i: (0, 0)),
    ],
    out_specs=pl.BlockSpec((_BLK, _H), lambda i: (i, 0)),
    out_shape=jax.ShapeDtypeStruct((_NP, _H), jnp.float32),
)


def _tc_head_body(h, wmu, bmu, wls, bls, wdec, bdec, xo, muo, lso):
    mu = lax.dot_general(h[...], wmu[...], (((1,), (1,)), ((), ())),
                         preferred_element_type=jnp.float32) + bmu[...]
    ls = lax.dot_general(h[...], wls[...], (((1,), (1,)), ((), ())),
                         preferred_element_type=jnp.float32) + bls[...]
    ls = jnp.minimum(ls, 10.0)
    xr = lax.dot_general(mu, wdec[...], (((1,), (1,)), ((), ())),
                         preferred_element_type=jnp.float32) + bdec[...]
    xo[...] = xr
    muo[...] = mu
    lso[...] = ls


_tc_head = pl.pallas_call(
    _tc_head_body,
    grid=(4,),
    in_specs=[
        pl.BlockSpec((_BLK, _H), lambda i: (i, 0)),
        pl.BlockSpec((_L, _H), lambda i: (0, 0)),
        pl.BlockSpec((1, _L), lambda i: (0, 0)),
        pl.BlockSpec((_L, _H), lambda i: (0, 0)),
        pl.BlockSpec((1, _L), lambda i: (0, 0)),
        pl.BlockSpec((_H, _L), lambda i: (0, 0)),
        pl.BlockSpec((1, _H), lambda i: (0, 0)),
    ],
    out_specs=[
        pl.BlockSpec((_BLK, _H), lambda i: (i, 0)),
        pl.BlockSpec((_BLK, _L), lambda i: (i, 0)),
        pl.BlockSpec((_BLK, _L), lambda i: (i, 0)),
    ],
    out_shape=[
        jax.ShapeDtypeStruct((_NP, _H), jnp.float32),
        jax.ShapeDtypeStruct((_NP, _L), jnp.float32),
        jax.ShapeDtypeStruct((_NP, _L), jnp.float32),
    ],
)


def kernel(x, edge_index, Wl0, bl0, Wr0, g0, be0, Wl1, bl1, Wr1, g1, be1,
           Wl2, bl2, Wr2, g2, be2, Wmu, bmu, Wls, bls, Wdec, bdec):
    src = edge_index[0]
    dst = edge_index[1]
    pad = jnp.full((_EP - _E,), _DUMMY, jnp.int32)
    src2d = jnp.concatenate([src, pad]).reshape(_EP // _CHUNK, _CHUNK)
    dst2d = jnp.concatenate([dst, pad]).reshape(_EP // _CHUNK, _CHUNK)

    zrows = jnp.zeros((_ROWS_PER_SUB, _H), jnp.float32)
    ones = jnp.ones((_CHUNK, _H), jnp.float32)

    cnt = _sc_cnt(dst2d, ones, zrows)
    c0 = cnt[0]
    c1 = cnt[1]

    h = jnp.pad(x, ((0, _NP - _N), (0, 0)))
    layers = [(Wl0, bl0, Wr0, g0, be0), (Wl1, bl1, Wr1, g1, be1),
              (Wl2, bl2, Wr2, g2, be2)]
    for (Wl, bl, Wr, g, be) in layers:
        agg = _sc_agg(h, src2d, dst2d, zrows)
        h = _tc_layer(agg[0], agg[1], c0, c1, h, Wl, bl.reshape(1, _H), Wr,
                      g.reshape(1, _H), be.reshape(1, _H))

    xr, mu, ls = _tc_head(h, Wmu, bmu.reshape(1, _L), Wls, bls.reshape(1, _L),
                          Wdec, bdec.reshape(1, _H))
    xr = xr[:_N]
    mu = mu[:_N]
    ls = ls[:_N]
    return (xr, mu, mu, ls)


# speedup vs baseline: 2.5521x; 1.0009x over previous
import math

import jax
import jax.numpy as jnp
from jax import lax
from jax.experimental import pallas as pl
from jax.experimental.pallas import tpu as pltpu
from jax.experimental.pallas import tpu_sc as plsc

_N = 10000
_NP = 10240
_E = 320000
_EP = 327680
_H = 128
_L = 64
_EPS = 1e-05
_TILES = 32
_CHUNK = 128
_NCHUNK = _EP // _TILES // _CHUNK
_ROWS_PER_SUB = _NP // 16
_DUMMY = 10008

_sc_mesh = plsc.VectorSubcoreMesh(core_axis_name="c", subcore_axis_name="s")

_GRP = 8
_CH_A = 152
_CH_B = 8


def _sc_agg_body(table, src2d, dst2d, zrows, out, agg_sh, sblk, dblk,
                 rows0, rows1, g0, g1):
    c = lax.axis_index("c")
    s = lax.axis_index("s")
    rows = (rows0, rows1)
    gsems = (g0, g1)
    nch = jnp.where(c == 0, _CH_A, _CH_B)
    base = jnp.where(c == 0, s * _CH_A, 16 * _CH_A + s * _CH_B)
    ngrp = nch // _GRP
    pltpu.sync_copy(zrows, agg_sh.at[pl.ds(s * _ROWS_PER_SUB, _ROWS_PER_SUB)])
    plsc.subcore_barrier()

    def grp(g, carry):
        pltpu.sync_copy(src2d.at[pl.ds(base + g * _GRP, _GRP)], sblk)
        pltpu.sync_copy(dst2d.at[pl.ds(base + g * _GRP, _GRP)], dblk)
        pltpu.async_copy(table.at[sblk.at[0]], rows0, g0)
        pltpu.async_copy(table.at[sblk.at[1]], rows1, g1)
        for j in range(_GRP):
            b = j % 2
            pltpu.make_async_copy(table.at[sblk.at[j]], rows[b],
                                  gsems[b]).wait()
            pltpu.sync_copy(rows[b], agg_sh.at[dblk.at[j]], add=True)
            if j + 2 < _GRP:
                pltpu.async_copy(table.at[sblk.at[j + 2]], rows[b], gsems[b])
        return carry

    lax.fori_loop(0, ngrp, grp, 0)
    plsc.subcore_barrier()
    pltpu.sync_copy(agg_sh.at[pl.ds(s * _ROWS_PER_SUB, _ROWS_PER_SUB)],
                    out.at[c, pl.ds(s * _ROWS_PER_SUB, _ROWS_PER_SUB)])


_sc_agg = pl.kernel(
    _sc_agg_body,
    out_type=jax.ShapeDtypeStruct((2, _NP, _H), jnp.float32),
    mesh=_sc_mesh,
    scratch_types=[
        pltpu.VMEM_SHARED((_NP, _H), jnp.float32),
        pltpu.VMEM((_GRP, _CHUNK), jnp.int32),
        pltpu.VMEM((_GRP, _CHUNK), jnp.int32),
        pltpu.VMEM((_CHUNK, _H), jnp.float32),
        pltpu.VMEM((_CHUNK, _H), jnp.float32),
        pltpu.SemaphoreType.DMA,
        pltpu.SemaphoreType.DMA,
    ],
)


def _sc_cnt_body(dst2d, ones, zrows, out, cnt_sh, dst_v, ones_v):
    c = lax.axis_index("c")
    s = lax.axis_index("s")
    wid = s * 2 + c
    pltpu.sync_copy(zrows, cnt_sh.at[pl.ds(s * _ROWS_PER_SUB, _ROWS_PER_SUB)])
    pltpu.sync_copy(dst2d.at[pl.ds(wid * _NCHUNK, _NCHUNK)], dst_v)
    pltpu.sync_copy(ones, ones_v)
    plsc.subcore_barrier()

    def chunk(i, carry):
        pltpu.sync_copy(ones_v, cnt_sh.at[dst_v.at[i]], add=True)
        return carry

    lax.fori_loop(0, _NCHUNK, chunk, 0)
    plsc.subcore_barrier()
    pltpu.sync_copy(cnt_sh.at[pl.ds(s * _ROWS_PER_SUB, _ROWS_PER_SUB)],
                    out.at[c, pl.ds(s * _ROWS_PER_SUB, _ROWS_PER_SUB)])


_sc_cnt = pl.kernel(
    _sc_cnt_body,
    out_type=jax.ShapeDtypeStruct((2, _NP, _H), jnp.float32),
    mesh=_sc_mesh,
    scratch_types=[
        pltpu.VMEM_SHARED((_NP, _H), jnp.float32),
        pltpu.VMEM((_NCHUNK, _CHUNK), jnp.int32),
        pltpu.VMEM((_CHUNK, _H), jnp.float32),
    ],
)


_BLK = 2560
_RSQRT2 = 1.0 / math.sqrt(2.0)
_BN_SCALE = 1.0 / math.sqrt(1.0 + _EPS)


def _tc_layer_body(p0, p1, c0, c1, h, wl, bl, wr, g, be, o):
    i = pl.program_id(0)
    cnt = c0[:, 0:1] + c1[:, 0:1]
    mean = (p0[...] + p1[...]) / jnp.maximum(cnt, 1.0)
    t = lax.dot_general(mean, wl[...], (((1,), (1,)), ((), ())),
                        preferred_element_type=jnp.float32)
    t = t + bl[...]
    t = t + lax.dot_general(h[...], wr[...], (((1,), (1,)), ((), ())),
                            preferred_element_type=jnp.float32)
    t = (t * _BN_SCALE) * g[...] + be[...]
    a = 0.5 * t * (1.0 + lax.erf(t * _RSQRT2))
    row = i * _BLK + lax.broadcasted_iota(jnp.int32, a.shape, 0)
    o[...] = jnp.where(row < _N, a, 0.0)


_tc_layer = pl.pallas_call(
    _tc_layer_body,
    grid=(4,),
    in_specs=[
        pl.BlockSpec((_BLK, _H), lambda i: (i, 0)),
        pl.BlockSpec((_BLK, _H), lambda i: (i, 0)),
        pl.BlockSpec((_BLK, _H), lambda i: (i, 0)),
        pl.BlockSpec((_BLK, _H), lambda i: (i, 0)),
        pl.BlockSpec((_BLK, _H), lambda i: (i, 0)),
        pl.BlockSpec((_H, _H), lambda i: (0, 0)),
        pl.BlockSpec((1, _H), lambda i: (0, 0)),
        pl.BlockSpec((_H, _H), lambda i: (0, 0)),
        pl.BlockSpec((1, _H), lambda i: (0, 0)),
        pl.BlockSpec((1, _H), lambda i: (0, 0)),
    ],
    out_specs=pl.BlockSpec((_BLK, _H), lambda i: (i, 0)),
    out_shape=jax.ShapeDtypeStruct((_NP, _H), jnp.float32),
)


def _tc_head_body(h, wmu, bmu, wls, bls, wdec, bdec, xo, muo, lso):
    mu = lax.dot_general(h[...], wmu[...], (((1,), (1,)), ((), ())),
                         preferred_element_type=jnp.float32) + bmu[...]
    ls = lax.dot_general(h[...], wls[...], (((1,), (1,)), ((), ())),
                         preferred_element_type=jnp.float32) + bls[...]
    ls = jnp.minimum(ls, 10.0)
    xr = lax.dot_general(mu, wdec[...], (((1,), (1,)), ((), ())),
                         preferred_element_type=jnp.float32) + bdec[...]
    xo[...] = xr
    muo[...] = mu
    lso[...] = ls


_tc_head = pl.pallas_call(
    _tc_head_body,
    grid=(4,),
    in_specs=[
        pl.BlockSpec((_BLK, _H), lambda i: (i, 0)),
        pl.BlockSpec((_L, _H), lambda i: (0, 0)),
        pl.BlockSpec((1, _L), lambda i: (0, 0)),
        pl.BlockSpec((_L, _H), lambda i: (0, 0)),
        pl.BlockSpec((1, _L), lambda i: (0, 0)),
        pl.BlockSpec((_H, _L), lambda i: (0, 0)),
        pl.BlockSpec((1, _H), lambda i: (0, 0)),
    ],
    out_specs=[
        pl.BlockSpec((_BLK, _H), lambda i: (i, 0)),
        pl.BlockSpec((_BLK, _L), lambda i: (i, 0)),
        pl.BlockSpec((_BLK, _L), lambda i: (i, 0)),
    ],
    out_shape=[
        jax.ShapeDtypeStruct((_NP, _H), jnp.float32),
        jax.ShapeDtypeStruct((_NP, _L), jnp.float32),
        jax.ShapeDtypeStruct((_NP, _L), jnp.float32),
    ],
)


def kernel(x, edge_index, Wl0, bl0, Wr0, g0, be0, Wl1, bl1, Wr1, g1, be1,
           Wl2, bl2, Wr2, g2, be2, Wmu, bmu, Wls, bls, Wdec, bdec):
    src = edge_index[0]
    dst = edge_index[1]
    pad = jnp.full((_EP - _E,), _DUMMY, jnp.int32)
    src2d = jnp.concatenate([src, pad]).reshape(_EP // _CHUNK, _CHUNK)
    dst2d = jnp.concatenate([dst, pad]).reshape(_EP // _CHUNK, _CHUNK)

    zrows = jnp.zeros((_ROWS_PER_SUB, _H), jnp.float32)
    ones = jnp.ones((_CHUNK, _H), jnp.float32)

    cnt = _sc_cnt(dst2d, ones, zrows)
    c0 = cnt[0]
    c1 = cnt[1]

    h = jnp.pad(x, ((0, _NP - _N), (0, 0)))
    layers = [(Wl0, bl0, Wr0, g0, be0), (Wl1, bl1, Wr1, g1, be1),
              (Wl2, bl2, Wr2, g2, be2)]
    for (Wl, bl, Wr, g, be) in layers:
        agg = _sc_agg(h, src2d, dst2d, zrows)
        h = _tc_layer(agg[0], agg[1], c0, c1, h, Wl, bl.reshape(1, _H), Wr,
                      g.reshape(1, _H), be.reshape(1, _H))

    xr, mu, ls = _tc_head(h, Wmu, bmu.reshape(1, _L), Wls, bls.reshape(1, _L),
                          Wdec, bdec.reshape(1, _H))
    xr = xr[:_N]
    mu = mu[:_N]
    ls = ls[:_N]
    return (xr, mu, mu, ls)
